# baseline (speedup 1.0000x reference)
import functools

import jax
import jax.numpy as jnp
from jax import lax
from jax.experimental import pallas as pl
from jax.experimental.pallas import tpu as pltpu
from jax.experimental.pallas import tpu_sc as plsc

NC = 2
NS = 16
NW = NC * NS
L = 16

B = 16384
D = 32
BPW = B // NW
CH = 128
NCH = BPW // CH

NE = 1000000
NR = 1000

EPW = 31232
ECH = 256
TAIL0 = NW * EPW

_mesh = plsc.VectorSubcoreMesh(core_axis_name="c", subcore_axis_name="s")

_params = pltpu.CompilerParams(use_tc_tiling_on_sc=True,
                               needs_layout_passes=False)


@functools.partial(
    pl.kernel,
    mesh=_mesh,
    out_type=jax.ShapeDtypeStruct((NE // 4, 128), jnp.float32),
    scratch_types=[
        pltpu.VMEM((D, ECH), jnp.float32),
        pltpu.VMEM((ECH // 4, 128), jnp.float32),
    ],
    compiler_params=_params,
)
def _detile_sc(ent_hbm, tail_hbm, out_hbm, fm, em):
    w = lax.axis_index("s") * NC + lax.axis_index("c")
    base = w * EPW

    iota = lax.iota(jnp.int32, L)

    def transpose_block(nents):
        for g in range(nents // L):
            ents = g * L + iota
            rows = lax.shift_right_logical(ents, 2)
            cols0 = lax.shift_left(lax.bitwise_and(ents, 3), 5)
            for d in range(D):
                vals = plsc.load_gather(
                    fm, [jnp.full((L,), d, jnp.int32), ents])
                plsc.store_scatter(em, [rows, cols0 + d], vals)

    def chunk(c, _):
        src0 = pl.multiple_of(base + c * ECH, 128)
        pltpu.sync_copy(ent_hbm.at[:, pl.ds(src0, ECH)], fm)
        transpose_block(ECH)
        dst0 = pl.multiple_of((base + c * ECH) // 4, 64)
        pltpu.sync_copy(em, out_hbm.at[pl.ds(dst0, ECH // 4)])
        return 0

    lax.fori_loop(0, EPW // ECH, chunk, 0)

    @pl.when(w == NW - 1)
    def _():
        for t in range(2):
            src0 = TAIL0 + t * ECH
            pltpu.sync_copy(ent_hbm.at[:, pl.ds(src0, ECH)], fm)
            transpose_block(ECH)
            pltpu.sync_copy(em, out_hbm.at[pl.ds(src0 // 4, ECH // 4)])
        pltpu.sync_copy(tail_hbm, em.at[pl.ds(0, 16)])
        pltpu.sync_copy(em.at[pl.ds(0, 16)],
                        out_hbm.at[pl.ds((TAIL0 + 512) // 4, 16)])


@functools.partial(
    pl.kernel,
    mesh=_mesh,
    out_type=jax.ShapeDtypeStruct((B // 4, 128), jnp.float32),
    scratch_types=[
        pltpu.VMEM((NCH, CH), jnp.int32),
        pltpu.VMEM((NCH, CH), jnp.int32),
        pltpu.VMEM((NCH, CH), jnp.int32),
        pltpu.VMEM((NCH, CH), jnp.int32),
        pltpu.VMEM((NCH, CH), jnp.int32),
        pltpu.VMEM((CH, 128), jnp.float32),
        pltpu.VMEM((CH, 128), jnp.float32),
        pltpu.VMEM((NR // 4, 128), jnp.float32),
        pltpu.VMEM((CH // 4, 128), jnp.float32),
        pltpu.SemaphoreType.DMA,
    ],
    compiler_params=_params,
)
def _gather_sc(e1_hbm, e2_hbm, r_hbm, ent4_hbm, rel_hbm, out_hbm,
               i1, i2, ir, s1, s2, g1, g2, relv, ob, sem):
    w = lax.axis_index("s") * NC + lax.axis_index("c")
    base = w * BPW

    pltpu.sync_copy(rel_hbm, relv)
    for j in range(NCH):
        off = pl.multiple_of(base + j * CH, 128)
        pltpu.sync_copy(e1_hbm.at[pl.ds(off, CH)], i1.at[j])
        pltpu.sync_copy(e2_hbm.at[pl.ds(off, CH)], i2.at[j])
        pltpu.sync_copy(r_hbm.at[pl.ds(off, CH)], ir.at[j])

    iota = lax.iota(jnp.int32, L)

    def prep(j, _):
        for l in range(CH // L):
            s = pl.ds(l * L, L)
            s1[j, s] = lax.shift_right_logical(i1[j, s], 2)
            s2[j, s] = lax.shift_right_logical(i2[j, s], 2)
        return 0

    lax.fori_loop(0, NCH, prep, 0)

    for j in range(NCH):
        c1 = pltpu.async_copy(ent4_hbm.at[s1.at[j]], g1, sem)
        c2 = pltpu.async_copy(ent4_hbm.at[s2.at[j]], g2, sem)
        c1.wait()
        c2.wait()

        def group(g, _):
            s = pl.ds(g * L, L)
            rowvec = g * L + iota
            sub1 = lax.shift_left(lax.bitwise_and(i1[j, s], 3), 5)
            sub2 = lax.shift_left(lax.bitwise_and(i2[j, s], 3), 5)
            rv_ = ir[j, s]
            rrow = lax.shift_right_logical(rv_, 2)
            rsub = lax.shift_left(lax.bitwise_and(rv_, 3), 5)
            bvec = g * L + iota
            brow = lax.shift_right_logical(bvec, 2)
            bsub = lax.shift_left(lax.bitwise_and(bvec, 3), 5)
            for d in range(D):
                a = plsc.load_gather(g1, [rowvec, sub1 + d])
                c = plsc.load_gather(g2, [rowvec, sub2 + d])
                rv = plsc.load_gather(relv, [rrow, rsub + d])
                plsc.store_scatter(ob, [brow, bsub + d], jnp.abs(a + rv - c))
            return 0

        lax.fori_loop(0, CH // L, group, 0)
        dst0 = pl.multiple_of((base + j * CH) // 4, 32)
        pltpu.sync_copy(ob, out_hbm.at[pl.ds(dst0, CH // 4)])


def kernel(e1, e2, r, entity_emb, rel_emb):
    tail = entity_emb[NE - 64:].reshape(16, 128)
    ent4 = _detile_sc(entity_emb.T, tail)
    rel4 = rel_emb.reshape(NR // 4, 128)
    out = _gather_sc(e1.astype(jnp.int32), e2.astype(jnp.int32),
                     r.astype(jnp.int32), ent4, rel4)
    return out.reshape(B, D)


# speedup vs baseline: 1.7411x; 1.7411x over previous
import functools

import jax
import jax.numpy as jnp
from jax import lax
from jax.experimental import pallas as pl
from jax.experimental.pallas import tpu as pltpu
from jax.experimental.pallas import tpu_sc as plsc

NC = 2
NS = 16
NW = NC * NS
L = 16

B = 16384
D = 32
BPW = B // NW
CH = 128
NCH = BPW // CH

_mesh = plsc.VectorSubcoreMesh(core_axis_name="c", subcore_axis_name="s")


@functools.partial(
    pl.kernel,
    mesh=_mesh,
    out_type=jax.ShapeDtypeStruct((NW, NCH, CH, D), jnp.float32),
    scratch_types=[
        pltpu.VMEM((NCH, CH), jnp.int32),
        pltpu.VMEM((NCH, CH), jnp.int32),
        pltpu.VMEM((NCH, CH), jnp.int32),
        pltpu.VMEM((NCH, CH, D), jnp.float32),
        pltpu.VMEM((NCH, CH, D), jnp.float32),
        pltpu.VMEM((NCH, CH, D), jnp.float32),
        pltpu.SemaphoreType.DMA,
    ],
    compiler_params=pltpu.CompilerParams(use_tc_tiling_on_sc=False),
)
def _ptranse_sc(e1_hbm, e2_hbm, r_hbm, ent_hbm, rel_hbm, out_hbm,
                i1, i2, ir, r1, r2, rr, sem):
    wid = lax.axis_index("s") * NC + lax.axis_index("c")

    pltpu.sync_copy(e1_hbm.at[wid], i1)
    pltpu.sync_copy(e2_hbm.at[wid], i2)
    pltpu.sync_copy(r_hbm.at[wid], ir)

    copies = []
    for j in range(NCH):
        copies.append(pltpu.async_copy(ent_hbm.at[i1.at[j]], r1.at[j], sem))
        copies.append(pltpu.async_copy(ent_hbm.at[i2.at[j]], r2.at[j], sem))
        copies.append(pltpu.async_copy(rel_hbm.at[ir.at[j]], rr.at[j], sem))
    for c in copies:
        c.wait()

    def row_body(i, _):
        for j in range(NCH):
            for h in range(D // L):
                s = pl.ds(h * L, L)
                r1[j, i, s] = jnp.abs(r1[j, i, s] + rr[j, i, s] - r2[j, i, s])
        return 0

    lax.fori_loop(0, CH, row_body, 0)

    pltpu.sync_copy(r1, out_hbm.at[wid])


def kernel(e1, e2, r, entity_emb, rel_emb):
    e1w = e1.astype(jnp.int32).reshape(NW, NCH, CH)
    e2w = e2.astype(jnp.int32).reshape(NW, NCH, CH)
    rw = r.astype(jnp.int32).reshape(NW, NCH, CH)
    out = _ptranse_sc(e1w, e2w, rw, entity_emb, rel_emb)
    return out.reshape(B, D)
